# baseline (speedup 1.0000x reference)
import functools

import jax
import jax.numpy as jnp
from jax import lax
from jax.experimental import pallas as pl
from jax.experimental.pallas import tpu as pltpu
from jax.experimental.pallas import tpu_sc as plsc

NUM_CLASSES = 100
CHANNELS = 64
H = 8
W = 8
BATCH = 1024
FEAT = CHANNELS * H * W
SUB = 32
LANE = 128

CLS_PAD = 104

NC = 2
NS = 16
NW = NC * NS
JC = 4
NCH = NUM_CLASSES // JC
MAX_ROWS = 4


def _tc_proto_body(x_ref, lg_ref, proto_ref):
    lg = lg_ref[...]
    m = jnp.max(lg, axis=1, keepdims=True)
    idx2 = lax.broadcasted_iota(jnp.int32, lg.shape, 1)
    cls = jnp.min(jnp.where(lg == m, idx2, NUM_CLASSES), axis=1)
    onehot = (cls[:, None] == lax.broadcasted_iota(
        jnp.int32, (BATCH, CLS_PAD), 1)).astype(jnp.float32)
    sums = lax.dot_general(
        onehot, x_ref[...],
        dimension_numbers=(((0,), (0,)), ((), ())),
        preferred_element_type=jnp.float32)
    counts = jnp.sum(onehot, axis=0)
    denom = jnp.where(counts > 0, counts, 1.0)
    proto = sums / denom[:, None]
    proto_ref[...] = proto[:NUM_CLASSES, :]


def _sc_diff_body(proto_hbm, out_hbm, pi_buf, pj_a, pj_b,
                  ob0, ob1, ob2, ob3, pi_sem, pja_sem, pjb_sem,
                  os0, os1, os2, os3):
    c = lax.axis_index("c")
    s = lax.axis_index("s")
    wid = s * NC + c
    r0 = (wid * NUM_CLASSES) // NW
    r1 = ((wid + 1) * NUM_CLASSES) // NW
    n = r1 - r0

    obs = [ob0, ob1, ob2, ob3]
    out_sems = [os0, os1, os2, os3]

    def pj_copy(ch, buf, sem):
        return pltpu.make_async_copy(
            proto_hbm.at[pl.ds(ch * JC, JC)], buf, sem)

    def do_chunk(ch, cur, first_static):
        for il in range(MAX_ROWS):
            ob = obs[il]

            @pl.when(il < n)
            def _piece(ob=ob, cur=cur, il=il):
                h_out = pltpu.make_async_copy(
                    ob, out_hbm.at[r0 + il, pl.ds(ch * JC, JC)], out_sems[il])
                if not first_static:
                    h_out.wait()

                def body(sb, carry):
                    for l in range(LANE // 16):
                        piv = pi_buf[il, sb, pl.ds(l * 16, 16)]
                        for jl in range(JC):
                            ob[jl, sb, pl.ds(l * 16, 16)] = (
                                cur[jl, sb, pl.ds(l * 16, 16)] - piv)
                    return carry

                lax.fori_loop(0, SUB, body, 0, unroll=2)
                h_out.start()

    h_pi = pltpu.make_async_copy(proto_hbm.at[pl.ds(r0, MAX_ROWS)], pi_buf, pi_sem)
    h_pi.start()
    pj_copy(0, pj_a, pja_sem).start()
    h_pi.wait()

    pj_copy(1, pj_b, pjb_sem).start()
    pj_copy(0, pj_a, pja_sem).wait()
    do_chunk(0, pj_a, True)

    def pair_body(t, carry):
        c1 = 2 * t + 1
        pj_copy(c1 + 1, pj_a, pja_sem).start()
        pj_copy(c1, pj_b, pjb_sem).wait()
        do_chunk(c1, pj_b, False)
        c2 = 2 * t + 2
        pj_copy(c2 + 1, pj_b, pjb_sem).start()
        pj_copy(c2, pj_a, pja_sem).wait()
        do_chunk(c2, pj_a, False)
        return carry

    lax.fori_loop(0, (NCH - 3) // 2, pair_body, 0)
    c1 = NCH - 2
    pj_copy(c1 + 1, pj_a, pja_sem).start()
    pj_copy(c1, pj_b, pjb_sem).wait()
    do_chunk(c1, pj_b, False)
    c2 = NCH - 1
    pj_copy(c2, pj_a, pja_sem).wait()
    do_chunk(c2, pj_a, False)

    for il in range(MAX_ROWS):
        @pl.when(il < n)
        def _drain(il=il):
            pltpu.make_async_copy(
                obs[il], out_hbm.at[r0 + il, pl.ds((NCH - 1) * JC, JC)],
                out_sems[il]).wait()


_sc_diff = functools.partial(
    pl.kernel,
    out_type=jax.ShapeDtypeStruct(
        (NUM_CLASSES, NUM_CLASSES, SUB, LANE), jnp.float32),
    mesh=plsc.VectorSubcoreMesh(core_axis_name="c", subcore_axis_name="s"),
    scratch_types=[
        pltpu.VMEM((MAX_ROWS, SUB, LANE), jnp.float32),
        pltpu.VMEM((JC, SUB, LANE), jnp.float32),
        pltpu.VMEM((JC, SUB, LANE), jnp.float32),
        pltpu.VMEM((JC, SUB, LANE), jnp.float32),
        pltpu.VMEM((JC, SUB, LANE), jnp.float32),
        pltpu.VMEM((JC, SUB, LANE), jnp.float32),
        pltpu.VMEM((JC, SUB, LANE), jnp.float32),
        pltpu.SemaphoreType.DMA,
        pltpu.SemaphoreType.DMA,
        pltpu.SemaphoreType.DMA,
        pltpu.SemaphoreType.DMA,
        pltpu.SemaphoreType.DMA,
        pltpu.SemaphoreType.DMA,
        pltpu.SemaphoreType.DMA,
    ],
)(_sc_diff_body)


def kernel(x, class_logits):
    xf = x.reshape(BATCH, FEAT)
    proto = pl.pallas_call(
        _tc_proto_body,
        in_specs=[
            pl.BlockSpec((BATCH, FEAT), lambda: (0, 0)),
            pl.BlockSpec((BATCH, NUM_CLASSES), lambda: (0, 0)),
        ],
        out_specs=pl.BlockSpec((NUM_CLASSES, FEAT), lambda: (0, 0)),
        out_shape=jax.ShapeDtypeStruct((NUM_CLASSES, FEAT), jnp.float32),
    )(xf, class_logits)
    inter = _sc_diff(proto.reshape(NUM_CLASSES, SUB, LANE))
    prototypes = proto.reshape(NUM_CLASSES, CHANNELS, H, W)
    inter_class_matrix = inter.reshape(NUM_CLASSES, NUM_CLASSES, CHANNELS, H, W)
    return (prototypes, inter_class_matrix)


# speedup vs baseline: 1.9488x; 1.9488x over previous
import functools

import jax
import jax.numpy as jnp
from jax import lax
from jax.experimental import pallas as pl
from jax.experimental.pallas import tpu as pltpu
from jax.experimental.pallas import tpu_sc as plsc

NUM_CLASSES = 100
CHANNELS = 64
H = 8
W = 8
BATCH = 1024
FEAT = CHANNELS * H * W
SUB = 32
LANE = 128

CLS_PAD = 104

NC = 2
NS = 16
NW = NC * NS
JC = 4
NCH = NUM_CLASSES // JC
MAX_ROWS = 4


def _tc_proto_body(x_ref, lg_ref, proto_ref):
    lg = lg_ref[...]
    m = jnp.max(lg, axis=1, keepdims=True)
    idx2 = lax.broadcasted_iota(jnp.int32, lg.shape, 1)
    cls = jnp.min(jnp.where(lg == m, idx2, NUM_CLASSES), axis=1)
    onehot = (cls[:, None] == lax.broadcasted_iota(
        jnp.int32, (BATCH, CLS_PAD), 1)).astype(jnp.float32)
    sums = lax.dot_general(
        onehot, x_ref[...],
        dimension_numbers=(((0,), (0,)), ((), ())),
        preferred_element_type=jnp.float32)
    counts = jnp.sum(onehot, axis=0)
    denom = jnp.where(counts > 0, counts, 1.0)
    proto = sums / denom[:, None]
    proto_ref[...] = proto[:NUM_CLASSES, :]


def _sc_diff_body(proto_hbm, out_hbm, pi_buf, pj_a, pj_b,
                  ob0, ob1, ob2, ob3, pi_sem, pja_sem, pjb_sem,
                  os0, os1, os2, os3):
    c = lax.axis_index("c")
    s = lax.axis_index("s")
    wid = s * NC + c
    r0 = (wid * NUM_CLASSES) // NW
    r1 = ((wid + 1) * NUM_CLASSES) // NW
    n = r1 - r0

    obs = [ob0, ob1, ob2, ob3]
    out_sems = [os0, os1, os2, os3]

    def pj_copy(ch, buf, sem):
        return pltpu.make_async_copy(
            proto_hbm.at[pl.ds(ch * JC, JC)], buf, sem)

    def do_chunk(ch, cur, first_static):
        for il in range(MAX_ROWS):
            ob = obs[il]

            @pl.when(il < n)
            def _piece(ob=ob, cur=cur, il=il):
                h_out = pltpu.make_async_copy(
                    ob, out_hbm.at[r0 + il, pl.ds(ch * JC, JC)], out_sems[il])
                if not first_static:
                    h_out.wait()

                def _sb_body(sb, carry):
                    piv = [pi_buf[il, sb, pl.ds(l * 16, 16)]
                           for l in range(LANE // 16)]
                    res = [cur[jl, sb, pl.ds(l * 16, 16)] - piv[l]
                           for jl in range(JC) for l in range(LANE // 16)]
                    k = 0
                    for jl in range(JC):
                        for l in range(LANE // 16):
                            ob[jl, sb, pl.ds(l * 16, 16)] = res[k]
                            k += 1
                    return carry

                lax.fori_loop(0, SUB, _sb_body, 0, unroll=2)
                h_out.start()

    h_pi = pltpu.make_async_copy(proto_hbm.at[pl.ds(r0, MAX_ROWS)], pi_buf, pi_sem)
    h_pi.start()
    pj_copy(0, pj_a, pja_sem).start()
    h_pi.wait()

    pj_copy(1, pj_b, pjb_sem).start()
    pj_copy(0, pj_a, pja_sem).wait()
    do_chunk(0, pj_a, True)

    def pair_body(t, carry):
        c1 = 2 * t + 1
        pj_copy(c1 + 1, pj_a, pja_sem).start()
        pj_copy(c1, pj_b, pjb_sem).wait()
        do_chunk(c1, pj_b, False)
        c2 = 2 * t + 2
        pj_copy(c2 + 1, pj_b, pjb_sem).start()
        pj_copy(c2, pj_a, pja_sem).wait()
        do_chunk(c2, pj_a, False)
        return carry

    lax.fori_loop(0, (NCH - 3) // 2, pair_body, 0)
    c1 = NCH - 2
    pj_copy(c1 + 1, pj_a, pja_sem).start()
    pj_copy(c1, pj_b, pjb_sem).wait()
    do_chunk(c1, pj_b, False)
    c2 = NCH - 1
    pj_copy(c2, pj_a, pja_sem).wait()
    do_chunk(c2, pj_a, False)

    for il in range(MAX_ROWS):
        @pl.when(il < n)
        def _drain(il=il):
            pltpu.make_async_copy(
                obs[il], out_hbm.at[r0 + il, pl.ds((NCH - 1) * JC, JC)],
                out_sems[il]).wait()


_sc_diff = functools.partial(
    pl.kernel,
    out_type=jax.ShapeDtypeStruct(
        (NUM_CLASSES, NUM_CLASSES, SUB, LANE), jnp.float32),
    mesh=plsc.VectorSubcoreMesh(core_axis_name="c", subcore_axis_name="s"),
    scratch_types=[
        pltpu.VMEM((MAX_ROWS, SUB, LANE), jnp.float32),
        pltpu.VMEM((JC, SUB, LANE), jnp.float32),
        pltpu.VMEM((JC, SUB, LANE), jnp.float32),
        pltpu.VMEM((JC, SUB, LANE), jnp.float32),
        pltpu.VMEM((JC, SUB, LANE), jnp.float32),
        pltpu.VMEM((JC, SUB, LANE), jnp.float32),
        pltpu.VMEM((JC, SUB, LANE), jnp.float32),
        pltpu.SemaphoreType.DMA,
        pltpu.SemaphoreType.DMA,
        pltpu.SemaphoreType.DMA,
        pltpu.SemaphoreType.DMA,
        pltpu.SemaphoreType.DMA,
        pltpu.SemaphoreType.DMA,
        pltpu.SemaphoreType.DMA,
    ],
)(_sc_diff_body)


def kernel(x, class_logits):
    xf = x.reshape(BATCH, FEAT)
    proto = pl.pallas_call(
        _tc_proto_body,
        in_specs=[
            pl.BlockSpec((BATCH, FEAT), lambda: (0, 0)),
            pl.BlockSpec((BATCH, NUM_CLASSES), lambda: (0, 0)),
        ],
        out_specs=pl.BlockSpec((NUM_CLASSES, FEAT), lambda: (0, 0)),
        out_shape=jax.ShapeDtypeStruct((NUM_CLASSES, FEAT), jnp.float32),
    )(xf, class_logits)
    inter = _sc_diff(proto.reshape(NUM_CLASSES, SUB, LANE))
    prototypes = proto.reshape(NUM_CLASSES, CHANNELS, H, W)
    inter_class_matrix = inter.reshape(NUM_CLASSES, NUM_CLASSES, CHANNELS, H, W)
    return (prototypes, inter_class_matrix)


# speedup vs baseline: 8.5432x; 4.3839x over previous
import functools

import jax
import jax.numpy as jnp
from jax import lax
from jax.experimental import pallas as pl
from jax.experimental.pallas import tpu as pltpu

NUM_CLASSES = 100
CHANNELS = 64
H = 8
W = 8
BATCH = 1024
FEAT = CHANNELS * H * W

CLS_PAD = 104
ROW_BLK = 8


def _tc_proto_body(xt_ref, lgt_ref, protot_ref, proto_ref):
    lg = lgt_ref[...]
    m = jnp.max(lg, axis=0, keepdims=True)
    idx2 = lax.broadcasted_iota(jnp.int32, lg.shape, 0)
    cls = jnp.min(jnp.where(lg == m, idx2, NUM_CLASSES), axis=0,
                  keepdims=True)
    onehot_t = (cls == lax.broadcasted_iota(
        jnp.int32, (CLS_PAD, BATCH), 0)).astype(jnp.float32)
    xt = xt_ref[...]
    sums_t = lax.dot_general(
        xt, onehot_t,
        dimension_numbers=(((1,), (1,)), ((), ())),
        preferred_element_type=jnp.float32)
    sums = lax.dot_general(
        onehot_t, xt,
        dimension_numbers=(((1,), (1,)), ((), ())),
        preferred_element_type=jnp.float32)
    counts = jnp.sum(onehot_t, axis=1)
    denom = jnp.where(counts > 0, counts, 1.0)
    protot_ref[...] = (sums_t / denom[None, :])[:, :NUM_CLASSES]
    proto_ref[...] = (sums / denom[:, None])[:NUM_CLASSES, :]


def _tc_diff_body(pt_ref, pi_ref, out_ref):
    pt = pt_ref[...]
    pi = pi_ref[...]
    out_ref[...] = pt[None, :, :] - pi[:, :, None]


def kernel(x, class_logits):
    xt = jnp.transpose(x, (1, 2, 3, 0)).reshape(FEAT, BATCH)
    lgt = jnp.transpose(class_logits, (1, 0))
    protot, proto = pl.pallas_call(
        _tc_proto_body,
        in_specs=[
            pl.BlockSpec((FEAT, BATCH), lambda: (0, 0)),
            pl.BlockSpec((NUM_CLASSES, BATCH), lambda: (0, 0)),
        ],
        out_specs=[
            pl.BlockSpec((FEAT, NUM_CLASSES), lambda: (0, 0)),
            pl.BlockSpec((NUM_CLASSES, FEAT), lambda: (0, 0)),
        ],
        out_shape=[
            jax.ShapeDtypeStruct((FEAT, NUM_CLASSES), jnp.float32),
            jax.ShapeDtypeStruct((NUM_CLASSES, FEAT), jnp.float32),
        ],
    )(xt, lgt)

    n_steps = pl.cdiv(NUM_CLASSES, ROW_BLK)
    inter = pl.pallas_call(
        _tc_diff_body,
        grid=(n_steps,),
        in_specs=[
            pl.BlockSpec((FEAT, NUM_CLASSES), lambda b: (0, 0)),
            pl.BlockSpec((ROW_BLK, FEAT), lambda b: (b, 0)),
        ],
        out_specs=pl.BlockSpec((ROW_BLK, FEAT, NUM_CLASSES), lambda b: (b, 0, 0)),
        out_shape=jax.ShapeDtypeStruct(
            (NUM_CLASSES, FEAT, NUM_CLASSES), jnp.float32),
    )(protot, proto)

    prototypes = jnp.transpose(
        protot.reshape(CHANNELS, H, W, NUM_CLASSES), (3, 0, 1, 2))
    inter_class_matrix = jnp.transpose(
        inter.reshape(NUM_CLASSES, CHANNELS, H, W, NUM_CLASSES),
        (0, 4, 1, 2, 3))
    return (prototypes, inter_class_matrix)
